# Initial kernel scaffold; baseline (speedup 1.0000x reference)
import functools

import jax
import jax.numpy as jnp
from jax import lax
from jax.experimental import pallas as pl
from jax.experimental.pallas import tpu as pltpu
from jax.experimental.pallas import tpu_sc as plsc

N = 10000
E = 320000
D = 128
NPAD = 10240
NC = 2
NS = 16
NW = NC * NS
K = 80
ROWS = E // K
RPT_E = ROWS // NW
RPT_H = ROWS // NS
SLICE = NPAD // NS


def _tc1_body(x_ref, w_ref, b_ref, xlin_ref):
    acc = lax.dot_general(x_ref[...], w_ref[...],
                          (((1,), (1,)), ((), ())),
                          preferred_element_type=jnp.float32)
    xlin_ref[...] = acc + b_ref[...]


def _tc1(x, W, b2):
    blk = 1000
    return pl.pallas_call(
        _tc1_body,
        grid=(N // blk,),
        in_specs=[
            pl.BlockSpec((blk, D), lambda i: (i, 0)),
            pl.BlockSpec((D, D), lambda i: (0, 0)),
            pl.BlockSpec((1, D), lambda i: (0, 0)),
        ],
        out_specs=pl.BlockSpec((blk, D), lambda i: (i, 0)),
        out_shape=jax.ShapeDtypeStruct((N, D), jnp.float32),
    )(x, W, b2)


def _sc_body(src_r, dst_r, a0r, a1r, a2r, xlin, be0, be1, be2,
             p0, p1, recip,
             acc, hist, snorm,
             srcb, dstb, eidxb, a1b, a2b,
             be0v, be1v, be2v, combv, normv,
             xrows, mrows, hbuf, nbuf, rbuf, onebuf, ones80):
    c = lax.axis_index("c")
    s = lax.axis_index("s")
    w = c * NS + s

    def fill_one(i, carry):
        onebuf[pl.ds(i * 16, 16)] = jnp.full((16,), 1.0, jnp.float32)
        return carry
    lax.fori_loop(0, SLICE // 16, fill_one, 0)

    def fill_one80(i, carry):
        ones80[pl.ds(i * 16, 16)] = jnp.full((16,), 1.0, jnp.float32)
        return carry
    lax.fori_loop(0, K // 16, fill_one80, 0)

    def fill_zero(e, carry):
        for g in range(D // 16):
            mrows[e, pl.ds(g * 16, 16)] = jnp.zeros((16,), jnp.float32)
        return carry
    lax.fori_loop(0, K, fill_zero, 0)

    pltpu.sync_copy(onebuf, hist.at[pl.ds(s * SLICE, SLICE)])
    for k2 in range(SLICE // K):
        pltpu.sync_copy(mrows, acc.at[pl.ds(s * SLICE + k2 * K, K)])
    plsc.subcore_barrier()

    pltpu.sync_copy(src_r.at[pl.ds(s * RPT_H, RPT_H)], srcb)

    def hist_loop(ch, carry):
        pltpu.sync_copy(ones80, hist.at[srcb.at[ch]], add=True)
        return carry
    lax.fori_loop(0, RPT_H, hist_loop, 0)
    plsc.subcore_barrier()

    pltpu.sync_copy(hist.at[pl.ds(s * SLICE, SLICE)], hbuf)

    def norm_loop(i, carry):
        d = hbuf[pl.ds(i * 16, 16)]
        ibits = lax.bitcast_convert_type(d, jnp.int32)
        y = lax.bitcast_convert_type(jnp.int32(0x5F3759DF) - (ibits >> 1),
                                     jnp.float32)
        for _ in range(3):
            y = y * (1.5 - 0.5 * d * y * y)
        nbuf[pl.ds(i * 16, 16)] = y
        rbuf[pl.ds(i * 16, 16)] = 1.0 / d
        return carry
    lax.fori_loop(0, SLICE // 16, norm_loop, 0)

    pltpu.sync_copy(nbuf, snorm.at[pl.ds(s * SLICE, SLICE)])

    @pl.when(c == 0)
    def _():
        pltpu.sync_copy(rbuf, recip.at[pl.ds(s * SLICE, SLICE)])

    plsc.subcore_barrier()
    pltpu.sync_copy(snorm, normv)

    pltpu.sync_copy(be0, be0v)
    pltpu.sync_copy(be1, be1v)
    pltpu.sync_copy(be2, be2v)

    def combo_loop(n, carry):
        i = n // 12
        r = n - i * 12
        j = r // 2
        k3 = r - j * 2
        for g in range(D // 16):
            sl = pl.ds(g * 16, 16)
            combv[n, sl] = be0v[i, sl] + be1v[j, sl] + be2v[k3, sl]
        return carry
    lax.fori_loop(0, 60, combo_loop, 0)

    base = w * RPT_E
    pltpu.sync_copy(src_r.at[pl.ds(base, RPT_E)], srcb.at[pl.ds(0, RPT_E)])
    pltpu.sync_copy(dst_r.at[pl.ds(base, RPT_E)], dstb)
    pltpu.sync_copy(a0r.at[pl.ds(base, RPT_E)], eidxb)
    pltpu.sync_copy(a1r.at[pl.ds(base, RPT_E)], a1b)
    pltpu.sync_copy(a2r.at[pl.ds(base, RPT_E)], a2b)

    def eidx_loop(r, carry):
        for g in range(K // 16):
            sl = pl.ds(g * 16, 16)
            eidxb[r, sl] = eidxb[r, sl] * 12 + a1b[r, sl] * 2 + a2b[r, sl]
        return carry
    lax.fori_loop(0, RPT_E, eidx_loop, 0)

    iota16 = lax.iota(jnp.int32, 16)

    def chunk_loop(ch, carry):
        pltpu.sync_copy(xlin.at[srcb.at[ch]], xrows)
        for g in range(K // 16):
            sl = pl.ds(g * 16, 16)
            sv = srcb[ch, sl]
            dv = dstb[ch, sl]
            ev = eidxb[ch, sl]
            scale = (plsc.load_gather(normv, [sv])
                     * plsc.load_gather(normv, [dv]))
            rows = iota16 + g * 16

            def col_loop(col, carry2):
                colv = jnp.full((16,), col, jnp.int32)
                xv = plsc.load_gather(xrows, [rows, colv])
                evv = plsc.load_gather(combv, [ev, colv])
                mv = jnp.maximum(xv + evv, 0.0) * scale
                plsc.store_scatter(mrows, [rows, colv], mv)
                return carry2
            lax.fori_loop(0, D, col_loop, 0)
        pltpu.sync_copy(mrows, acc.at[dstb.at[ch]], add=True)
        return carry
    lax.fori_loop(0, RPT_E, chunk_loop, 0)
    plsc.subcore_barrier()

    @pl.when(c == 0)
    def _():
        pltpu.sync_copy(acc.at[pl.ds(s * SLICE, SLICE)],
                        p0.at[pl.ds(s * SLICE, SLICE)])

    @pl.when(c == 1)
    def _():
        pltpu.sync_copy(acc.at[pl.ds(s * SLICE, SLICE)],
                        p1.at[pl.ds(s * SLICE, SLICE)])


_sc_call = pl.kernel(
    _sc_body,
    out_type=(
        jax.ShapeDtypeStruct((NPAD, D), jnp.float32),
        jax.ShapeDtypeStruct((NPAD, D), jnp.float32),
        jax.ShapeDtypeStruct((NPAD,), jnp.float32),
    ),
    mesh=plsc.VectorSubcoreMesh(core_axis_name="c", subcore_axis_name="s"),
    scratch_types=[
        pltpu.VMEM_SHARED((NPAD, D), jnp.float32),
        pltpu.VMEM_SHARED((NPAD,), jnp.float32),
        pltpu.VMEM_SHARED((NPAD,), jnp.float32),
        pltpu.VMEM((RPT_H, K), jnp.int32),
        pltpu.VMEM((RPT_E, K), jnp.int32),
        pltpu.VMEM((RPT_E, K), jnp.int32),
        pltpu.VMEM((RPT_E, K), jnp.int32),
        pltpu.VMEM((RPT_E, K), jnp.int32),
        pltpu.VMEM((5, D), jnp.float32),
        pltpu.VMEM((6, D), jnp.float32),
        pltpu.VMEM((2, D), jnp.float32),
        pltpu.VMEM((60, D), jnp.float32),
        pltpu.VMEM((NPAD,), jnp.float32),
        pltpu.VMEM((K, D), jnp.float32),
        pltpu.VMEM((K, D), jnp.float32),
        pltpu.VMEM((SLICE,), jnp.float32),
        pltpu.VMEM((SLICE,), jnp.float32),
        pltpu.VMEM((SLICE,), jnp.float32),
        pltpu.VMEM((SLICE,), jnp.float32),
        pltpu.VMEM((K,), jnp.float32),
    ],
)


def _tc2_body(p0_ref, p1_ref, xlin_ref, root_ref, rec_ref, out_ref):
    self_term = jnp.maximum(xlin_ref[...] + root_ref[...], 0.0) * rec_ref[...]
    out_ref[...] = p0_ref[...] + p1_ref[...] + self_term


def _tc2(p0, p1, xlin, root_emb, rec2):
    blk = 400
    return pl.pallas_call(
        _tc2_body,
        grid=(N // blk,),
        in_specs=[
            pl.BlockSpec((blk, D), lambda i: (i, 0)),
            pl.BlockSpec((blk, D), lambda i: (i, 0)),
            pl.BlockSpec((blk, D), lambda i: (i, 0)),
            pl.BlockSpec((1, D), lambda i: (0, 0)),
            pl.BlockSpec((blk, 1), lambda i: (i, 0)),
        ],
        out_specs=pl.BlockSpec((blk, D), lambda i: (i, 0)),
        out_shape=jax.ShapeDtypeStruct((N, D), jnp.float32),
    )(p0, p1, xlin, root_emb, rec2)


def kernel(x, edge_index, edge_attr, W, b, root_emb, be0, be1, be2):
    src_r = edge_index[0].reshape(ROWS, K)
    dst_r = edge_index[1].reshape(ROWS, K)
    a0r = edge_attr[:, 0].reshape(ROWS, K)
    a1r = edge_attr[:, 1].reshape(ROWS, K)
    a2r = edge_attr[:, 2].reshape(ROWS, K)

    xlin = _tc1(x, W, b.reshape(1, D))
    p0, p1, recip = _sc_call(src_r, dst_r, a0r, a1r, a2r, xlin, be0, be1, be2)
    return _tc2(p0, p1, xlin, root_emb, recip.reshape(NPAD, 1))


# speedup vs baseline: 3.2637x; 3.2637x over previous
import jax
import jax.numpy as jnp
from jax import lax
from jax.experimental import pallas as pl
from jax.experimental.pallas import tpu as pltpu
from jax.experimental.pallas import tpu_sc as plsc

N = 10000
E = 320000
D = 128
NPAD = 10240
NC = 2
NS = 16
NW = NC * NS
K = 80
SEG = 400
CPS = SEG // K
EPT_E = E // NW
EPT_H = E // NS
NSEG_E = EPT_E // SEG
NSEG_H = EPT_H // SEG
SLICE = NPAD // NS


def _tc1_body(x_ref, w_ref, b_ref, xlin_ref):
    acc = lax.dot_general(x_ref[...], w_ref[...],
                          (((1,), (1,)), ((), ())),
                          preferred_element_type=jnp.float32)
    xlin_ref[...] = acc + b_ref[...]


def _tc1(x, W, b2):
    blk = 1000
    return pl.pallas_call(
        _tc1_body,
        grid=(N // blk,),
        in_specs=[
            pl.BlockSpec((blk, D), lambda i: (i, 0)),
            pl.BlockSpec((D, D), lambda i: (0, 0)),
            pl.BlockSpec((1, D), lambda i: (0, 0)),
        ],
        out_specs=pl.BlockSpec((blk, D), lambda i: (i, 0)),
        out_shape=jax.ShapeDtypeStruct((N, D), jnp.float32),
    )(x, W, b2)


def _sc_body(src, dst, a0, a1, a2, xlin, be0, be1, be2,
             p0, p1, recip,
             acc, norms,
             segsrc, segdst, sega0, sega1, sega2, idxs2,
             be0v, be1v, be2v, combv, normv,
             xrows, mrows, hbuf, nbuf, rbuf, ones80):
    c = lax.axis_index("c")
    s = lax.axis_index("s")
    w = c * NS + s

    def fill_one(i, carry):
        hbuf[pl.ds(i * 16, 16)] = jnp.full((16,), 1.0, jnp.float32)
        return carry
    lax.fori_loop(0, SLICE // 16, fill_one, 0)

    def fill_one80(i, carry):
        ones80[pl.ds(i * 16, 16)] = jnp.full((16,), 1.0, jnp.float32)
        return carry
    lax.fori_loop(0, K // 16, fill_one80, 0)

    def fill_zero(e, carry):
        for g in range(D // 16):
            mrows[e, pl.ds(g * 16, 16)] = jnp.zeros((16,), jnp.float32)
        return carry
    lax.fori_loop(0, K, fill_zero, 0)

    pltpu.sync_copy(hbuf, norms.at[pl.ds(s * SLICE, SLICE)])
    for k2 in range(SLICE // K):
        pltpu.sync_copy(mrows, acc.at[pl.ds(s * SLICE + k2 * K, K)])
    plsc.subcore_barrier()

    def hist_seg(sg, carry):
        pltpu.sync_copy(src.at[pl.ds(s * EPT_H + sg * SEG, SEG)], segsrc)
        for ch in range(CPS):
            for g in range(K // 16):
                idxs2[0, pl.ds(g * 16, 16)] = (
                    segsrc[pl.ds(ch * K + g * 16, 16)])
            pltpu.sync_copy(ones80, norms.at[idxs2.at[0]], add=True)
        return carry
    lax.fori_loop(0, NSEG_H, hist_seg, 0)
    plsc.subcore_barrier()

    pltpu.sync_copy(norms.at[pl.ds(s * SLICE, SLICE)], hbuf)

    def norm_loop(i, carry):
        d = hbuf[pl.ds(i * 16, 16)]
        ibits = lax.bitcast_convert_type(d, jnp.int32)
        y = lax.bitcast_convert_type(jnp.int32(0x5F3759DF) - (ibits >> 1),
                                     jnp.float32)
        for _ in range(3):
            y = y * (1.5 - 0.5 * d * y * y)
        nbuf[pl.ds(i * 16, 16)] = y
        rbuf[pl.ds(i * 16, 16)] = 1.0 / d
        return carry
    lax.fori_loop(0, SLICE // 16, norm_loop, 0)

    pltpu.sync_copy(nbuf, norms.at[pl.ds(s * SLICE, SLICE)])

    @pl.when(c == 0)
    def _():
        pltpu.sync_copy(rbuf, recip.at[pl.ds(s * SLICE, SLICE)])

    plsc.subcore_barrier()
    pltpu.sync_copy(norms, normv)

    pltpu.sync_copy(be0, be0v)
    pltpu.sync_copy(be1, be1v)
    pltpu.sync_copy(be2, be2v)

    def combo_loop(n, carry):
        i = n // 12
        r = n - i * 12
        j = r // 2
        k3 = r - j * 2
        for g in range(D // 16):
            sl = pl.ds(g * 16, 16)
            combv[n, sl] = be0v[i, sl] + be1v[j, sl] + be2v[k3, sl]
        return carry
    lax.fori_loop(0, 60, combo_loop, 0)

    iota16 = lax.iota(jnp.int32, 16)

    def edge_seg(sg, carry):
        base = w * EPT_E + sg * SEG
        pltpu.sync_copy(src.at[pl.ds(base, SEG)], segsrc)
        pltpu.sync_copy(dst.at[pl.ds(base, SEG)], segdst)
        pltpu.sync_copy(a0.at[pl.ds(base, SEG)], sega0)
        pltpu.sync_copy(a1.at[pl.ds(base, SEG)], sega1)
        pltpu.sync_copy(a2.at[pl.ds(base, SEG)], sega2)
        for ch in range(CPS):
            evs = []
            scales = []
            for g in range(K // 16):
                sl = pl.ds(g * 16, 16)
                esl = pl.ds(ch * K + g * 16, 16)
                sv = segsrc[esl]
                dv = segdst[esl]
                evs.append(sega0[esl] * 12 + sega1[esl] * 2 + sega2[esl])
                idxs2[0, sl] = sv
                idxs2[1, sl] = dv
                scales.append(plsc.load_gather(normv, [sv])
                              * plsc.load_gather(normv, [dv]))
            pltpu.sync_copy(xlin.at[idxs2.at[0]], xrows)
            for g in range(K // 16):
                ev = evs[g]
                scale = scales[g]
                rows = iota16 + g * 16

                def col_loop(col, carry2):
                    colv = jnp.full((16,), col, jnp.int32)
                    xv = plsc.load_gather(xrows, [rows, colv])
                    evv = plsc.load_gather(combv, [ev, colv])
                    mv = jnp.maximum(xv + evv, 0.0) * scale
                    plsc.store_scatter(mrows, [rows, colv], mv)
                    return carry2
                lax.fori_loop(0, D, col_loop, 0)
            pltpu.sync_copy(mrows, acc.at[idxs2.at[1]], add=True)
        return carry
    lax.fori_loop(0, NSEG_E, edge_seg, 0)
    plsc.subcore_barrier()

    @pl.when(c == 0)
    def _():
        pltpu.sync_copy(acc.at[pl.ds(s * SLICE, SLICE)],
                        p0.at[pl.ds(s * SLICE, SLICE)])

    @pl.when(c == 1)
    def _():
        pltpu.sync_copy(acc.at[pl.ds(s * SLICE, SLICE)],
                        p1.at[pl.ds(s * SLICE, SLICE)])


_sc_call = pl.kernel(
    _sc_body,
    out_type=(
        jax.ShapeDtypeStruct((NPAD, D), jnp.float32),
        jax.ShapeDtypeStruct((NPAD, D), jnp.float32),
        jax.ShapeDtypeStruct((NPAD,), jnp.float32),
    ),
    mesh=plsc.VectorSubcoreMesh(core_axis_name="c", subcore_axis_name="s"),
    compiler_params=pltpu.CompilerParams(needs_layout_passes=False),
    scratch_types=[
        pltpu.VMEM_SHARED((NPAD, D), jnp.float32),
        pltpu.VMEM_SHARED((NPAD,), jnp.float32),
        pltpu.VMEM((SEG,), jnp.int32),
        pltpu.VMEM((SEG,), jnp.int32),
        pltpu.VMEM((SEG,), jnp.int32),
        pltpu.VMEM((SEG,), jnp.int32),
        pltpu.VMEM((SEG,), jnp.int32),
        pltpu.VMEM((2, K), jnp.int32),
        pltpu.VMEM((5, D), jnp.float32),
        pltpu.VMEM((6, D), jnp.float32),
        pltpu.VMEM((2, D), jnp.float32),
        pltpu.VMEM((60, D), jnp.float32),
        pltpu.VMEM((NPAD,), jnp.float32),
        pltpu.VMEM((K, D), jnp.float32),
        pltpu.VMEM((K, D), jnp.float32),
        pltpu.VMEM((SLICE,), jnp.float32),
        pltpu.VMEM((SLICE,), jnp.float32),
        pltpu.VMEM((SLICE,), jnp.float32),
        pltpu.VMEM((K,), jnp.float32),
    ],
)


def _tc2_body(p0_ref, p1_ref, xlin_ref, root_ref, rec_ref, out_ref):
    self_term = jnp.maximum(xlin_ref[...] + root_ref[...], 0.0) * rec_ref[...]
    out_ref[...] = p0_ref[...] + p1_ref[...] + self_term


def _tc2(p0, p1, xlin, root_emb, rec2):
    blk = 400
    return pl.pallas_call(
        _tc2_body,
        grid=(N // blk,),
        in_specs=[
            pl.BlockSpec((blk, D), lambda i: (i, 0)),
            pl.BlockSpec((blk, D), lambda i: (i, 0)),
            pl.BlockSpec((blk, D), lambda i: (i, 0)),
            pl.BlockSpec((1, D), lambda i: (0, 0)),
            pl.BlockSpec((blk, 1), lambda i: (i, 0)),
        ],
        out_specs=pl.BlockSpec((blk, D), lambda i: (i, 0)),
        out_shape=jax.ShapeDtypeStruct((N, D), jnp.float32),
    )(p0, p1, xlin, root_emb, rec2)


def kernel(x, edge_index, edge_attr, W, b, root_emb, be0, be1, be2):
    src = edge_index[0]
    dst = edge_index[1]
    a0 = edge_attr[:, 0]
    a1 = edge_attr[:, 1]
    a2 = edge_attr[:, 2]

    xlin = _tc1(x, W, b.reshape(1, D))
    p0, p1, recip = _sc_call(src, dst, a0, a1, a2, xlin, be0, be1, be2)
    return _tc2(p0, p1, xlin, root_emb, recip.reshape(NPAD, 1))
